# baseline (speedup 1.0000x reference)
import functools

import jax
import jax.numpy as jnp
from jax import lax
from jax.experimental import pallas as pl
from jax.experimental.pallas import tpu as pltpu
from jax.experimental.pallas import tpu_sc as plsc

T = 16384
D = 64
NCORES = 2
NSUB = 16
NW = NCORES * NSUB
TPW = T // NW
CHUNK = 128
NCHUNK = TPW // CHUNK


def _sc_gather(x_domain, a_head, a_tail, a_pids, constant_table,
               predicate_table):
  mesh = plsc.VectorSubcoreMesh(core_axis_name="c", subcore_axis_name="s")

  @functools.partial(
      pl.kernel,
      out_type=(
          jax.ShapeDtypeStruct((T, D), jnp.float32),
          jax.ShapeDtypeStruct((T, D), jnp.float32),
          jax.ShapeDtypeStruct((T, D), jnp.float32),
      ),
      mesh=mesh,
      compiler_params=pltpu.CompilerParams(needs_layout_passes=False),
      scratch_types=[
          pltpu.VMEM((TPW,), jnp.int32),
          pltpu.VMEM((TPW,), jnp.int32),
          pltpu.VMEM((TPW,), jnp.int32),
          pltpu.VMEM((TPW,), jnp.int32),
          pltpu.VMEM((TPW,), jnp.int32),
          pltpu.VMEM((CHUNK, D), jnp.float32),
          pltpu.VMEM((CHUNK, D), jnp.float32),
          pltpu.VMEM((CHUNK, D), jnp.float32),
          pltpu.VMEM((CHUNK, D), jnp.float32),
          pltpu.VMEM((CHUNK, D), jnp.float32),
          pltpu.VMEM((CHUNK, D), jnp.float32),
          pltpu.SemaphoreType.DMA,
          pltpu.SemaphoreType.DMA,
          pltpu.SemaphoreType.DMA,
      ],
  )
  def k(xdom_hbm, ah_hbm, at_hbm, ap_hbm, ctab_hbm, ptab_hbm,
        p_out, h_out, t_out,
        hidx_v, tidx_v, pidx_v, chidx_v, ctidx_v,
        hrows0_v, trows0_v, prows0_v, hrows1_v, trows1_v, prows1_v,
        sem_g, sem_r0, sem_r1):
    wid = lax.axis_index("s") * NCORES + lax.axis_index("c")
    base0 = wid * TPW
    rows = ((hrows0_v, trows0_v, prows0_v), (hrows1_v, trows1_v, prows1_v))
    sems = (sem_r0, sem_r1)
    pltpu.sync_copy(ah_hbm.at[pl.ds(base0, TPW)], hidx_v)
    pltpu.sync_copy(at_hbm.at[pl.ds(base0, TPW)], tidx_v)
    pltpu.sync_copy(ap_hbm.at[pl.ds(base0, TPW)], pidx_v)
    composes = []
    for c in range(NCHUNK):
      sl = pl.ds(c * CHUNK, CHUNK)
      composes.append(
          pltpu.async_copy(xdom_hbm.at[hidx_v.at[sl]], chidx_v.at[sl], sem_g))
      composes.append(
          pltpu.async_copy(xdom_hbm.at[tidx_v.at[sl]], ctidx_v.at[sl], sem_g))
    for cp in composes:
      cp.wait()

    def fire(c, buf):
      hrows_v, trows_v, prows_v = rows[buf]
      sem = sems[buf]

      def fire_group(j, _):
        g = c * CHUNK + j * 16
        o = j * 16
        ch16 = chidx_v[pl.ds(g, 16)]
        ct16 = ctidx_v[pl.ds(g, 16)]
        pi16 = pidx_v[pl.ds(g, 16)]
        for l in range(16):
          pltpu.async_copy(ctab_hbm.at[pl.ds(ch16[l], 1), :],
                           hrows_v.at[pl.ds(o + l, 1), :], sem)
          pltpu.async_copy(ctab_hbm.at[pl.ds(ct16[l], 1), :],
                           trows_v.at[pl.ds(o + l, 1), :], sem)
          pltpu.async_copy(ptab_hbm.at[pl.ds(pi16[l], 1), :],
                           prows_v.at[pl.ds(o + l, 1), :], sem)
        return _

      lax.fori_loop(0, CHUNK // 16, fire_group, None)

    def drain_and_store(c, buf):
      hrows_v, trows_v, prows_v = rows[buf]
      sem = sems[buf]

      def drain(i, _):
        pltpu.make_async_copy(ctab_hbm.at[pl.ds(0, 1), :],
                              hrows_v.at[pl.ds(0, 1), :], sem).wait()
        pltpu.make_async_copy(ctab_hbm.at[pl.ds(0, 1), :],
                              trows_v.at[pl.ds(0, 1), :], sem).wait()
        pltpu.make_async_copy(ptab_hbm.at[pl.ds(0, 1), :],
                              prows_v.at[pl.ds(0, 1), :], sem).wait()
        return _

      lax.fori_loop(0, CHUNK, drain, None)
      base = base0 + c * CHUNK
      pltpu.sync_copy(hrows_v, h_out.at[pl.ds(base, CHUNK)])
      pltpu.sync_copy(trows_v, t_out.at[pl.ds(base, CHUNK)])
      pltpu.sync_copy(prows_v, p_out.at[pl.ds(base, CHUNK)])

    fire(0, 0)
    for c in range(1, NCHUNK):
      fire(c, c % 2)
      drain_and_store(c - 1, (c - 1) % 2)
    drain_and_store(NCHUNK - 1, (NCHUNK - 1) % 2)

  return k(x_domain, a_head, a_tail, a_pids, constant_table, predicate_table)


VOCAB = 1000000
TBLK = 32768


def _tc_transpose(ct_t):
  grid = (VOCAB + TBLK - 1) // TBLK
  return pl.pallas_call(
      lambda x_ref, o_ref: o_ref.__setitem__(
          (slice(None), slice(None)), x_ref[...].T),
      grid=(grid,),
      in_specs=[pl.BlockSpec((D, TBLK), lambda i: (0, i))],
      out_specs=pl.BlockSpec((TBLK, D), lambda i: (i, 0)),
      out_shape=jax.ShapeDtypeStruct((VOCAB, D), jnp.float32),
      compiler_params=pltpu.CompilerParams(
          vmem_limit_bytes=110 * 1024 * 1024),
  )(ct_t)


BT = 4096


def _tc_mlp_body(p_ref, h_ref, t_ref, w1_ref, b1_ref, wot_ref, bo_ref,
                 embt_ref, out_ref):
  atom_in = jnp.concatenate([p_ref[...], h_ref[...], t_ref[...]], axis=1)
  acc = jnp.dot(atom_in, w1_ref[...], preferred_element_type=jnp.float32)
  emb_t = jnp.tanh(acc + b1_ref[...]).T
  embt_ref[...] = emb_t
  logit = jnp.dot(wot_ref[...], emb_t, preferred_element_type=jnp.float32)
  out_ref[...] = jax.nn.sigmoid(logit + bo_ref[...])


def _tc_mlp(p_emb, h_emb, t_emb, W1, b1, Wout_t, b_out):
  b1_2d = b1.reshape(1, D)
  bo_2d = b_out.reshape(1, 1)
  row_spec = pl.BlockSpec((BT, D), lambda i: (i, 0))
  full = lambda shape: pl.BlockSpec(shape, lambda i: (0,) * len(shape))
  emb_t, out_t = pl.pallas_call(
      _tc_mlp_body,
      grid=(T // BT,),
      in_specs=[
          row_spec, row_spec, row_spec,
          full((3 * D, D)), full((1, D)), full((1, D)), full((1, 1)),
      ],
      out_specs=[pl.BlockSpec((D, BT), lambda i: (0, i)),
                 pl.BlockSpec((1, BT), lambda i: (0, i))],
      out_shape=[
          jax.ShapeDtypeStruct((D, T), jnp.float32),
          jax.ShapeDtypeStruct((1, T), jnp.float32),
      ],
  )(p_emb, h_emb, t_emb, W1, b1_2d, Wout_t, bo_2d)
  return emb_t, out_t


def kernel(X_domain, A_idx, A_pids, constant_table, predicate_table, W1, b1,
           W_out, b_out):
  a_head = A_idx[:, 0]
  a_tail = A_idx[:, 1]
  ct_row = _tc_transpose(constant_table.T)
  p_emb, h_emb, t_emb = _sc_gather(
      X_domain, a_head, a_tail, A_pids, ct_row, predicate_table)
  emb_t, out_t = _tc_mlp(p_emb, h_emb, t_emb, W1, b1, W_out.T, b_out)
  return out_t.T[:, :, None], emb_t.T


# speedup vs baseline: 1.0046x; 1.0046x over previous
import functools

import jax
import jax.numpy as jnp
from jax import lax
from jax.experimental import pallas as pl
from jax.experimental.pallas import tpu as pltpu
from jax.experimental.pallas import tpu_sc as plsc

T = 16384
D = 64
NCORES = 2
NSUB = 16
NW = NCORES * NSUB
TPW = T // NW
CHUNK = 128
NCHUNK = TPW // CHUNK


def _sc_gather(x_domain, a_head, a_tail, a_pids, constant_table,
               predicate_table):
  mesh = plsc.VectorSubcoreMesh(core_axis_name="c", subcore_axis_name="s")

  @functools.partial(
      pl.kernel,
      out_type=(
          jax.ShapeDtypeStruct((T, D), jnp.float32),
          jax.ShapeDtypeStruct((T, D), jnp.float32),
          jax.ShapeDtypeStruct((T, D), jnp.float32),
      ),
      mesh=mesh,
      compiler_params=pltpu.CompilerParams(needs_layout_passes=False),
      scratch_types=[
          pltpu.VMEM((TPW,), jnp.int32),
          pltpu.VMEM((TPW,), jnp.int32),
          pltpu.VMEM((TPW,), jnp.int32),
          pltpu.VMEM((TPW,), jnp.int32),
          pltpu.VMEM((TPW,), jnp.int32),
          pltpu.VMEM((CHUNK, D), jnp.float32),
          pltpu.VMEM((CHUNK, D), jnp.float32),
          pltpu.VMEM((CHUNK, D), jnp.float32),
          pltpu.VMEM((CHUNK, D), jnp.float32),
          pltpu.VMEM((CHUNK, D), jnp.float32),
          pltpu.VMEM((CHUNK, D), jnp.float32),
          pltpu.SemaphoreType.DMA,
          pltpu.SemaphoreType.DMA,
          pltpu.SemaphoreType.DMA,
      ],
  )
  def k(xdom_hbm, ah_hbm, at_hbm, ap_hbm, ctab_hbm, ptab_hbm,
        p_out, h_out, t_out,
        hidx_v, tidx_v, pidx_v, chidx_v, ctidx_v,
        hrows0_v, trows0_v, prows0_v, hrows1_v, trows1_v, prows1_v,
        sem_g, sem_r0, sem_r1):
    wid = lax.axis_index("s") * NCORES + lax.axis_index("c")
    base0 = wid * TPW
    rows = ((hrows0_v, trows0_v, prows0_v), (hrows1_v, trows1_v, prows1_v))
    sems = (sem_r0, sem_r1)
    pltpu.sync_copy(ah_hbm.at[pl.ds(base0, TPW)], hidx_v)
    pltpu.sync_copy(at_hbm.at[pl.ds(base0, TPW)], tidx_v)
    pltpu.sync_copy(ap_hbm.at[pl.ds(base0, TPW)], pidx_v)
    composes = []
    for c in range(NCHUNK):
      sl = pl.ds(c * CHUNK, CHUNK)
      composes.append(
          pltpu.async_copy(xdom_hbm.at[hidx_v.at[sl]], chidx_v.at[sl], sem_g))
      composes.append(
          pltpu.async_copy(xdom_hbm.at[tidx_v.at[sl]], ctidx_v.at[sl], sem_g))
    for cp in composes:
      cp.wait()

    def fire(c, buf):
      hrows_v, trows_v, prows_v = rows[buf]
      sem = sems[buf]

      def fire_group(j, _):
        g = c * CHUNK + j * 16
        o = j * 16
        ch16 = chidx_v[pl.ds(g, 16)]
        ct16 = ctidx_v[pl.ds(g, 16)]
        pi16 = pidx_v[pl.ds(g, 16)]
        for l in range(16):
          pltpu.async_copy(ctab_hbm.at[pl.ds(ch16[l], 1), :],
                           hrows_v.at[pl.ds(o + l, 1), :], sem)
          pltpu.async_copy(ctab_hbm.at[pl.ds(ct16[l], 1), :],
                           trows_v.at[pl.ds(o + l, 1), :], sem)
          pltpu.async_copy(ptab_hbm.at[pl.ds(pi16[l], 1), :],
                           prows_v.at[pl.ds(o + l, 1), :], sem)
        return _

      lax.fori_loop(0, CHUNK // 16, fire_group, None)

    def drain_and_store(c, buf):
      hrows_v, trows_v, prows_v = rows[buf]
      sem = sems[buf]

      def drain(i, _):
        pltpu.make_async_copy(ctab_hbm.at[pl.ds(0, 1), :],
                              hrows_v.at[pl.ds(0, 1), :], sem).wait()
        pltpu.make_async_copy(ctab_hbm.at[pl.ds(0, 1), :],
                              trows_v.at[pl.ds(0, 1), :], sem).wait()
        pltpu.make_async_copy(ptab_hbm.at[pl.ds(0, 1), :],
                              prows_v.at[pl.ds(0, 1), :], sem).wait()
        return _

      lax.fori_loop(0, CHUNK, drain, None)
      base = base0 + c * CHUNK
      pltpu.sync_copy(hrows_v, h_out.at[pl.ds(base, CHUNK)])
      pltpu.sync_copy(trows_v, t_out.at[pl.ds(base, CHUNK)])
      pltpu.sync_copy(prows_v, p_out.at[pl.ds(base, CHUNK)])

    fire(0, 0)
    for c in range(1, NCHUNK):
      fire(c, c % 2)
      drain_and_store(c - 1, (c - 1) % 2)
    drain_and_store(NCHUNK - 1, (NCHUNK - 1) % 2)

  return k(x_domain, a_head, a_tail, a_pids, constant_table, predicate_table)


VOCAB = 1000000
TBLK = 40960


def _tc_transpose(ct_t):
  grid = (VOCAB + TBLK - 1) // TBLK
  return pl.pallas_call(
      lambda x_ref, o_ref: o_ref.__setitem__(
          (slice(None), slice(None)), x_ref[...].T),
      grid=(grid,),
      in_specs=[pl.BlockSpec((D, TBLK), lambda i: (0, i))],
      out_specs=pl.BlockSpec((TBLK, D), lambda i: (i, 0)),
      out_shape=jax.ShapeDtypeStruct((VOCAB, D), jnp.float32),
      compiler_params=pltpu.CompilerParams(
          vmem_limit_bytes=110 * 1024 * 1024),
  )(ct_t)


BT = 4096


def _tc_mlp_body(p_ref, h_ref, t_ref, w1_ref, b1_ref, wot_ref, bo_ref,
                 embt_ref, out_ref):
  atom_in = jnp.concatenate([p_ref[...], h_ref[...], t_ref[...]], axis=1)
  acc = jnp.dot(atom_in, w1_ref[...], preferred_element_type=jnp.float32)
  emb_t = jnp.tanh(acc + b1_ref[...]).T
  embt_ref[...] = emb_t
  logit = jnp.dot(wot_ref[...], emb_t, preferred_element_type=jnp.float32)
  out_ref[...] = jax.nn.sigmoid(logit + bo_ref[...])


def _tc_mlp(p_emb, h_emb, t_emb, W1, b1, Wout_t, b_out):
  b1_2d = b1.reshape(1, D)
  bo_2d = b_out.reshape(1, 1)
  row_spec = pl.BlockSpec((BT, D), lambda i: (i, 0))
  full = lambda shape: pl.BlockSpec(shape, lambda i: (0,) * len(shape))
  emb_t, out_t = pl.pallas_call(
      _tc_mlp_body,
      grid=(T // BT,),
      in_specs=[
          row_spec, row_spec, row_spec,
          full((3 * D, D)), full((1, D)), full((1, D)), full((1, 1)),
      ],
      out_specs=[pl.BlockSpec((D, BT), lambda i: (0, i)),
                 pl.BlockSpec((1, BT), lambda i: (0, i))],
      out_shape=[
          jax.ShapeDtypeStruct((D, T), jnp.float32),
          jax.ShapeDtypeStruct((1, T), jnp.float32),
      ],
  )(p_emb, h_emb, t_emb, W1, b1_2d, Wout_t, bo_2d)
  return emb_t, out_t


def kernel(X_domain, A_idx, A_pids, constant_table, predicate_table, W1, b1,
           W_out, b_out):
  a_head = A_idx[:, 0]
  a_tail = A_idx[:, 1]
  ct_row = _tc_transpose(constant_table.T)
  p_emb, h_emb, t_emb = _sc_gather(
      X_domain, a_head, a_tail, A_pids, ct_row, predicate_table)
  emb_t, out_t = _tc_mlp(p_emb, h_emb, t_emb, W1, b1, W_out.T, b_out)
  return out_t.T[:, :, None], emb_t.T
